# baseline (speedup 1.0000x reference)
import functools

import jax
import jax.numpy as jnp
from jax import lax
from jax.experimental import pallas as pl
from jax.experimental.pallas import tpu as pltpu
from jax.experimental.pallas import tpu_sc as plsc

SEQ = 2048
D = 768
WDIM = 20
WPAD = 48
NWIN = 8
NSPAN = NWIN * SEQ - (NWIN * (NWIN - 1)) // 2
OUTD = 2 * D + WDIM
R = 32
CHUNKS_PER_WORKER = 16


def _build():
    info = plsc.get_sparse_core_info()
    nc = info.num_cores

    mesh = plsc.VectorSubcoreMesh(core_axis_name="c", subcore_axis_name="s")

    @functools.partial(
        pl.kernel,
        mesh=mesh,
        out_type=jax.ShapeDtypeStruct((NSPAN, OUTD), jnp.float32),
        scratch_types=[
            pltpu.VMEM((R, OUTD), jnp.float32),
            pltpu.VMEM((R, OUTD), jnp.float32),
            pltpu.VMEM((WPAD,), jnp.float32),
            pltpu.SemaphoreType.DMA,
            pltpu.SemaphoreType.DMA,
            pltpu.SemaphoreType.DMA,
            pltpu.SemaphoreType.DMA,
        ],
        compiler_params=pltpu.CompilerParams(use_tc_tiling_on_sc=False),
    )
    def k(x_hbm, swe_hbm, out_hbm, buf_a, buf_b, wbuf,
          in_sem_a, in_sem_b, out_sem_a, out_sem_b):
        cid = lax.axis_index("c")
        sid = lax.axis_index("s")
        wid = sid * nc + cid
        wi = wid // 4
        q = wid % 4
        n = SEQ - wi
        off = SEQ * wi - (wi * (wi - 1)) // 2
        b = wi + 1 - (wi >= 5).astype(jnp.int32)

        pltpu.sync_copy(swe_hbm.at[pl.ds(b * WPAD, WPAD)], wbuf)
        w0 = wbuf[0:16]
        w1s = wbuf[24:40]
        for buf in (buf_a, buf_b):
            for r in range(R):
                buf[r, pl.ds(2 * D, 16)] = w0
                buf[r, pl.ds(2 * D + 4, 16)] = w1s

        outs = [None, None]
        for t in range(CHUNKS_PER_WORKER):
            par = t % 2
            buf = (buf_a, buf_b)[par]
            in_sem = (in_sem_a, in_sem_b)[par]
            out_sem = (out_sem_a, out_sem_b)[par]
            c = q * CHUNKS_PER_WORKER + t
            j0 = jnp.minimum(c * R, n - R)
            j1 = j0 + wi
            r0 = off + j0
            if outs[par] is not None:
                outs[par].wait()
            in0 = pltpu.make_async_copy(
                x_hbm.at[pl.ds(j0, R), :], buf.at[:, pl.ds(0, D)], in_sem)
            in1 = pltpu.make_async_copy(
                x_hbm.at[pl.ds(j1, R), :], buf.at[:, pl.ds(D, D)], in_sem)
            in0.start()
            in1.start()
            in0.wait()
            in1.wait()
            oc = pltpu.make_async_copy(
                buf, out_hbm.at[pl.ds(r0, R), :], out_sem)
            oc.start()
            outs[par] = oc
        outs[0].wait()
        outs[1].wait()

    return k


def kernel(x, span_width_embedding, batch_max_seq_len):
    del batch_max_seq_len
    x2 = x.reshape(SEQ, D)
    swe_flat = (
        jnp.zeros((span_width_embedding.shape[0], WPAD), span_width_embedding.dtype)
        .at[:, :WDIM]
        .set(span_width_embedding)
        .at[:, 24:40]
        .set(span_width_embedding[:, 4:20])
        .reshape(-1)
    )
    out = _build()(x2, swe_flat)
    return out.reshape(1, NSPAN, OUTD)
